# baseline (speedup 1.0000x reference)
import functools

import jax
import jax.numpy as jnp
from jax import lax
from jax.experimental import pallas as pl
from jax.experimental.pallas import tpu as pltpu
from jax.experimental.pallas import tpu_sc as plsc

N_NODES = 10000
D = 128
N_EDGES = 320000

NC = 2
NS = 16
NP = 10240
CHUNK = 128
TOT_CHUNKS = 2560
RING = 40
TILE_CHUNKS = TOT_CHUNKS // (NC * NS)
DEG_CHUNKS = TILE_CHUNKS
EP = TOT_CHUNKS * CHUNK
ROWS_PER_TILE = NP // NS

_mesh = plsc.VectorSubcoreMesh(core_axis_name="c", subcore_axis_name="s")


@functools.partial(
    pl.kernel,
    out_type=jax.ShapeDtypeStruct((NC, NP), jnp.float32),
    mesh=_mesh,
    scratch_types=[
        pltpu.VMEM((DEG_CHUNKS, CHUNK), jnp.int32),
        pltpu.VMEM((CHUNK,), jnp.float32),
        pltpu.VMEM_SHARED((NP,), jnp.float32),
        pltpu.SemaphoreType.DMA,
    ],
)
def _deg_kernel(dst_hbm, zeros1_hbm, degp_hbm, dstv, ones_v, deg_sp, dsem):
    cid = lax.axis_index("c")
    sid = lax.axis_index("s")
    wid = cid * NS + sid
    pltpu.sync_copy(dst_hbm.at[pl.ds(wid * DEG_CHUNKS, DEG_CHUNKS)], dstv)
    pltpu.sync_copy(
        zeros1_hbm.at[pl.ds(sid * ROWS_PER_TILE, ROWS_PER_TILE)],
        deg_sp.at[pl.ds(sid * ROWS_PER_TILE, ROWS_PER_TILE)],
    )
    for k in range(CHUNK // 16):
        ones_v[pl.ds(k * 16, 16)] = jnp.full((16,), 1.0, jnp.float32)
    plsc.subcore_barrier()

    def body(j, carry):
        pltpu.async_copy(ones_v, deg_sp.at[dstv.at[j]], dsem, add=True)
        return carry

    lax.fori_loop(0, DEG_CHUNKS, body, 0)

    def drain(j, carry):
        pltpu.make_async_copy(ones_v, deg_sp.at[dstv.at[j]], dsem).wait()
        return carry

    lax.fori_loop(0, DEG_CHUNKS, drain, 0)
    plsc.subcore_barrier()
    pltpu.sync_copy(
        deg_sp.at[pl.ds(sid * ROWS_PER_TILE, ROWS_PER_TILE)],
        degp_hbm.at[cid, pl.ds(sid * ROWS_PER_TILE, ROWS_PER_TILE)],
    )


@functools.partial(
    pl.kernel,
    out_type=jax.ShapeDtypeStruct((NC, NP, D), jnp.float32),
    mesh=_mesh,
    scratch_types=[
        pltpu.VMEM((RING, CHUNK), jnp.int32),
        pltpu.VMEM((RING, CHUNK), jnp.int32),
        pltpu.VMEM((CHUNK, D), jnp.float32),
        pltpu.VMEM((CHUNK, D), jnp.float32),
        pltpu.VMEM_SHARED((NP, D), jnp.float32),
        pltpu.SemaphoreType.DMA,
        pltpu.SemaphoreType.DMA,
    ],
)
def _msg_kernel(src_hbm, dst_hbm, g2_hbm, zeros2_hbm, outp_hbm,
                srcv, dstv, buf0, buf1, acc_sp, sem0, sem1):
    cid = lax.axis_index("c")
    sid = lax.axis_index("s")
    g_hbm = g2_hbm.at[cid]
    chunk0 = (cid * NS + sid) * TILE_CHUNKS
    pltpu.sync_copy(
        zeros2_hbm.at[pl.ds(sid * ROWS_PER_TILE, ROWS_PER_TILE)],
        acc_sp.at[pl.ds(sid * ROWS_PER_TILE, ROWS_PER_TILE)],
    )
    plsc.subcore_barrier()

    def ring_body(s, carry):
        base = chunk0 + s * RING
        pltpu.sync_copy(src_hbm.at[pl.ds(base, RING)], srcv)
        pltpu.sync_copy(dst_hbm.at[pl.ds(base, RING)], dstv)
        pltpu.make_async_copy(g_hbm.at[srcv.at[0]], buf0, sem0).start()

        def body(j, inner):
            c0 = 2 * j
            c1 = 2 * j + 1
            pltpu.make_async_copy(g_hbm.at[srcv.at[c1]], buf1, sem1).start()
            pltpu.make_async_copy(g_hbm.at[srcv.at[c0]], buf0, sem0).wait()
            pltpu.sync_copy(buf0, acc_sp.at[dstv.at[c0]], add=True)

            @pl.when(c0 + 2 < RING)
            def _():
                pltpu.make_async_copy(g_hbm.at[srcv.at[c0 + 2]], buf0,
                                      sem0).start()

            pltpu.make_async_copy(g_hbm.at[srcv.at[c1]], buf1, sem1).wait()
            pltpu.sync_copy(buf1, acc_sp.at[dstv.at[c1]], add=True)
            return inner

        lax.fori_loop(0, RING // 2, body, 0)
        return carry

    lax.fori_loop(0, TILE_CHUNKS // RING, ring_body, 0)
    plsc.subcore_barrier()
    pltpu.sync_copy(
        acc_sp.at[pl.ds(sid * ROWS_PER_TILE, ROWS_PER_TILE)],
        outp_hbm.at[cid, pl.ds(sid * ROWS_PER_TILE, ROWS_PER_TILE)],
    )


_BLK = 1024
_GRID = NP // _BLK


def _matmul_body(x_ref, w_ref, h_ref):
    h_ref[...] = jnp.dot(x_ref[...], w_ref[...],
                         preferred_element_type=jnp.float32)


def _matmul_call(x_pad, w):
    return pl.pallas_call(
        _matmul_body,
        grid=(_GRID,),
        in_specs=[
            pl.BlockSpec((_BLK, D), lambda i: (i, 0)),
            pl.BlockSpec((D, D), lambda i: (0, 0)),
        ],
        out_specs=pl.BlockSpec((_BLK, D), lambda i: (i, 0)),
        out_shape=jax.ShapeDtypeStruct((NP, D), jnp.float32),
    )(x_pad, w)


def _scale_body(h_ref, d0_ref, d1_ref, g2_ref, dinv_ref):
    deg = d0_ref[...] + d1_ref[...] + 1.0
    dinv = lax.rsqrt(deg)
    g = h_ref[...] * dinv
    g2_ref[0] = g
    g2_ref[1] = g
    dinv_ref[...] = dinv


def _scale_call(h, d0, d1):
    return pl.pallas_call(
        _scale_body,
        grid=(_GRID,),
        in_specs=[
            pl.BlockSpec((_BLK, D), lambda i: (i, 0)),
            pl.BlockSpec((_BLK, 1), lambda i: (i, 0)),
            pl.BlockSpec((_BLK, 1), lambda i: (i, 0)),
        ],
        out_specs=[
            pl.BlockSpec((NC, _BLK, D), lambda i: (0, i, 0)),
            pl.BlockSpec((_BLK, 1), lambda i: (i, 0)),
        ],
        out_shape=[
            jax.ShapeDtypeStruct((NC, NP, D), jnp.float32),
            jax.ShapeDtypeStruct((NP, 1), jnp.float32),
        ],
    )(h, d0, d1)


def _combine_body(p_ref, g_ref, dinv_ref, b_ref, o_ref):
    s = (p_ref[0] + p_ref[1] + g_ref[...]) * dinv_ref[...]
    o_ref[...] = jnp.maximum(s + b_ref[...], 0.0)


_CBLK = 1000


def _combine_call(partials, g, dinv, b2):
    return pl.pallas_call(
        _combine_body,
        grid=(N_NODES // _CBLK,),
        in_specs=[
            pl.BlockSpec((NC, _CBLK, D), lambda i: (0, i, 0)),
            pl.BlockSpec((_CBLK, D), lambda i: (i, 0)),
            pl.BlockSpec((_CBLK, 1), lambda i: (i, 0)),
            pl.BlockSpec((1, D), lambda i: (0, 0)),
        ],
        out_specs=pl.BlockSpec((_CBLK, D), lambda i: (i, 0)),
        out_shape=jax.ShapeDtypeStruct((N_NODES, D), jnp.float32),
    )(partials, g, dinv, b2)


def kernel(x, edge_index, W, b):
    src = edge_index[0].astype(jnp.int32)
    dst = edge_index[1].astype(jnp.int32)
    pad = EP - N_EDGES
    k = jnp.arange(pad, dtype=jnp.int32)
    trash_src = N_NODES + k % (NP - N_NODES)
    src_p = jnp.concatenate([src, trash_src])
    dst_p = jnp.concatenate([dst, k % NP])
    src2 = src_p.reshape(TOT_CHUNKS, CHUNK)
    dst2 = dst_p.reshape(TOT_CHUNKS, CHUNK)
    dstd_p = jnp.concatenate([dst, trash_src])
    dstd2 = dstd_p.reshape(TOT_CHUNKS, CHUNK)

    zeros1 = jnp.zeros((NP,), jnp.float32)
    zeros2 = jnp.zeros((NP, D), jnp.float32)

    degp = _deg_kernel(dstd2, zeros1)

    x_pad = jnp.concatenate([x, jnp.zeros((NP - N_NODES, D), x.dtype)], axis=0)
    h = _matmul_call(x_pad, W)
    d0 = degp[0].reshape(NP, 1)
    d1 = degp[1].reshape(NP, 1)
    g2, dinv = _scale_call(h, d0, d1)

    partials = _msg_kernel(src2, dst2, g2, zeros2)

    return _combine_call(partials, g2[0], dinv, b.reshape(1, D))


# speedup vs baseline: 1.0195x; 1.0195x over previous
import functools

import jax
import jax.numpy as jnp
from jax import lax
from jax.experimental import pallas as pl
from jax.experimental.pallas import tpu as pltpu
from jax.experimental.pallas import tpu_sc as plsc

N_NODES = 10000
D = 128
N_EDGES = 320000

NC = 2
NS = 16
NP = 10240
CHUNK = 128
TOT_CHUNKS = 2560
RING = 40
TILE_CHUNKS = TOT_CHUNKS // (NC * NS)
DEG_CHUNKS = TILE_CHUNKS
EP = TOT_CHUNKS * CHUNK
ROWS_PER_TILE = NP // NS

_mesh = plsc.VectorSubcoreMesh(core_axis_name="c", subcore_axis_name="s")


@functools.partial(
    pl.kernel,
    out_type=jax.ShapeDtypeStruct((NC, NP), jnp.float32),
    mesh=_mesh,
    scratch_types=[
        pltpu.VMEM((DEG_CHUNKS, CHUNK), jnp.int32),
        pltpu.VMEM((CHUNK,), jnp.float32),
        pltpu.VMEM_SHARED((NP,), jnp.float32),
        pltpu.SemaphoreType.DMA,
    ],
)
def _deg_kernel(dst_hbm, zeros1_hbm, degp_hbm, dstv, ones_v, deg_sp, dsem):
    cid = lax.axis_index("c")
    sid = lax.axis_index("s")
    wid = cid * NS + sid
    pltpu.sync_copy(dst_hbm.at[pl.ds(wid * DEG_CHUNKS, DEG_CHUNKS)], dstv)
    pltpu.sync_copy(
        zeros1_hbm.at[pl.ds(sid * ROWS_PER_TILE, ROWS_PER_TILE)],
        deg_sp.at[pl.ds(sid * ROWS_PER_TILE, ROWS_PER_TILE)],
    )
    for k in range(CHUNK // 16):
        ones_v[pl.ds(k * 16, 16)] = jnp.full((16,), 1.0, jnp.float32)
    plsc.subcore_barrier()

    def body(j, carry):
        pltpu.async_copy(ones_v, deg_sp.at[dstv.at[j]], dsem, add=True)
        return carry

    lax.fori_loop(0, DEG_CHUNKS, body, 0)

    def drain(j, carry):
        pltpu.make_async_copy(ones_v, deg_sp.at[dstv.at[j]], dsem).wait()
        return carry

    lax.fori_loop(0, DEG_CHUNKS, drain, 0)
    plsc.subcore_barrier()
    pltpu.sync_copy(
        deg_sp.at[pl.ds(sid * ROWS_PER_TILE, ROWS_PER_TILE)],
        degp_hbm.at[cid, pl.ds(sid * ROWS_PER_TILE, ROWS_PER_TILE)],
    )


@functools.partial(
    pl.kernel,
    out_type=jax.ShapeDtypeStruct((NC, NP, D), jnp.float32),
    mesh=_mesh,
    scratch_types=[
        pltpu.VMEM((RING, CHUNK), jnp.int32),
        pltpu.VMEM((RING, CHUNK), jnp.int32),
        pltpu.VMEM((CHUNK, D), jnp.float32),
        pltpu.VMEM((CHUNK, D), jnp.float32),
        pltpu.VMEM_SHARED((NP, D), jnp.float32),
        pltpu.SemaphoreType.DMA,
        pltpu.SemaphoreType.DMA,
    ],
)
def _msg_kernel(src_hbm, dst_hbm, g2_hbm, zeros2_hbm, outp_hbm,
                srcv, dstv, buf0, buf1, acc_sp, sem0, sem1):
    cid = lax.axis_index("c")
    sid = lax.axis_index("s")
    g_hbm = g2_hbm
    chunk0 = (cid * NS + sid) * TILE_CHUNKS
    pltpu.sync_copy(
        zeros2_hbm.at[pl.ds(sid * ROWS_PER_TILE, ROWS_PER_TILE)],
        acc_sp.at[pl.ds(sid * ROWS_PER_TILE, ROWS_PER_TILE)],
    )
    plsc.subcore_barrier()

    def ring_body(s, carry):
        base = chunk0 + s * RING
        pltpu.sync_copy(src_hbm.at[pl.ds(base, RING)], srcv)
        pltpu.sync_copy(dst_hbm.at[pl.ds(base, RING)], dstv)
        pltpu.make_async_copy(g_hbm.at[srcv.at[0]], buf0, sem0).start()

        def body(j, inner):
            c0 = 2 * j
            c1 = 2 * j + 1
            pltpu.make_async_copy(g_hbm.at[srcv.at[c1]], buf1, sem1).start()
            pltpu.make_async_copy(g_hbm.at[srcv.at[c0]], buf0, sem0).wait()
            pltpu.sync_copy(buf0, acc_sp.at[dstv.at[c0]], add=True)

            @pl.when(c0 + 2 < RING)
            def _():
                pltpu.make_async_copy(g_hbm.at[srcv.at[c0 + 2]], buf0,
                                      sem0).start()

            pltpu.make_async_copy(g_hbm.at[srcv.at[c1]], buf1, sem1).wait()
            pltpu.sync_copy(buf1, acc_sp.at[dstv.at[c1]], add=True)
            return inner

        lax.fori_loop(0, RING // 2, body, 0)
        return carry

    lax.fori_loop(0, TILE_CHUNKS // RING, ring_body, 0)
    plsc.subcore_barrier()
    pltpu.sync_copy(
        acc_sp.at[pl.ds(sid * ROWS_PER_TILE, ROWS_PER_TILE)],
        outp_hbm.at[cid, pl.ds(sid * ROWS_PER_TILE, ROWS_PER_TILE)],
    )


_BLK = 1024
_GRID = NP // _BLK


def _matmul_body(x_ref, w_ref, h_ref):
    h_ref[...] = jnp.dot(x_ref[...], w_ref[...],
                         preferred_element_type=jnp.float32)


def _matmul_call(x_pad, w):
    return pl.pallas_call(
        _matmul_body,
        grid=(_GRID,),
        in_specs=[
            pl.BlockSpec((_BLK, D), lambda i: (i, 0)),
            pl.BlockSpec((D, D), lambda i: (0, 0)),
        ],
        out_specs=pl.BlockSpec((_BLK, D), lambda i: (i, 0)),
        out_shape=jax.ShapeDtypeStruct((NP, D), jnp.float32),
    )(x_pad, w)


def _scale_body(h_ref, d0_ref, d1_ref, g2_ref, dinv_ref):
    deg = d0_ref[...] + d1_ref[...] + 1.0
    dinv = lax.rsqrt(deg)
    g2_ref[...] = h_ref[...] * dinv
    dinv_ref[...] = dinv


def _scale_call(h, d0, d1):
    return pl.pallas_call(
        _scale_body,
        grid=(_GRID,),
        in_specs=[
            pl.BlockSpec((_BLK, D), lambda i: (i, 0)),
            pl.BlockSpec((_BLK, 1), lambda i: (i, 0)),
            pl.BlockSpec((_BLK, 1), lambda i: (i, 0)),
        ],
        out_specs=[
            pl.BlockSpec((_BLK, D), lambda i: (i, 0)),
            pl.BlockSpec((_BLK, 1), lambda i: (i, 0)),
        ],
        out_shape=[
            jax.ShapeDtypeStruct((NP, D), jnp.float32),
            jax.ShapeDtypeStruct((NP, 1), jnp.float32),
        ],
    )(h, d0, d1)


def _combine_body(p_ref, g_ref, dinv_ref, b_ref, o_ref):
    s = (p_ref[0] + p_ref[1] + g_ref[...]) * dinv_ref[...]
    o_ref[...] = jnp.maximum(s + b_ref[...], 0.0)


_CBLK = 1000


def _combine_call(partials, g, dinv, b2):
    return pl.pallas_call(
        _combine_body,
        grid=(N_NODES // _CBLK,),
        in_specs=[
            pl.BlockSpec((NC, _CBLK, D), lambda i: (0, i, 0)),
            pl.BlockSpec((_CBLK, D), lambda i: (i, 0)),
            pl.BlockSpec((_CBLK, 1), lambda i: (i, 0)),
            pl.BlockSpec((1, D), lambda i: (0, 0)),
        ],
        out_specs=pl.BlockSpec((_CBLK, D), lambda i: (i, 0)),
        out_shape=jax.ShapeDtypeStruct((N_NODES, D), jnp.float32),
    )(partials, g, dinv, b2)


def kernel(x, edge_index, W, b):
    src = edge_index[0].astype(jnp.int32)
    dst = edge_index[1].astype(jnp.int32)
    pad = EP - N_EDGES
    k = jnp.arange(pad, dtype=jnp.int32)
    trash_src = N_NODES + k % (NP - N_NODES)
    src_p = jnp.concatenate([src, trash_src])
    dst_p = jnp.concatenate([dst, k % NP])
    src2 = src_p.reshape(TOT_CHUNKS, CHUNK)
    dst2 = dst_p.reshape(TOT_CHUNKS, CHUNK)
    dstd_p = jnp.concatenate([dst, trash_src])
    dstd2 = dstd_p.reshape(TOT_CHUNKS, CHUNK)

    zeros1 = jnp.zeros((NP,), jnp.float32)
    zeros2 = jnp.zeros((NP, D), jnp.float32)

    degp = _deg_kernel(dstd2, zeros1)

    x_pad = jnp.concatenate([x, jnp.zeros((NP - N_NODES, D), x.dtype)], axis=0)
    h = _matmul_call(x_pad, W)
    d0 = degp[0].reshape(NP, 1)
    d1 = degp[1].reshape(NP, 1)
    g2, dinv = _scale_call(h, d0, d1)

    partials = _msg_kernel(src2, dst2, g2, zeros2)

    return _combine_call(partials, g2, dinv, b.reshape(1, D))


# speedup vs baseline: 1.0490x; 1.0289x over previous
import functools

import jax
import jax.numpy as jnp
from jax import lax
from jax.experimental import pallas as pl
from jax.experimental.pallas import tpu as pltpu
from jax.experimental.pallas import tpu_sc as plsc

N_NODES = 10000
D = 128
N_EDGES = 320000

NC = 2
NS = 16
NPD = 10240
CHUNK = 125
TOT_CHUNKS = 2560
RING = 40
TILE_CHUNKS = TOT_CHUNKS // (NC * NS)
DEG_CHUNKS = TILE_CHUNKS
RPT = 632
DROWS_PER_TILE = NPD // NS

_mesh = plsc.VectorSubcoreMesh(core_axis_name="c", subcore_axis_name="s")


@functools.partial(
    pl.kernel,
    out_type=jax.ShapeDtypeStruct((NC, NPD), jnp.float32),
    mesh=_mesh,
    scratch_types=[
        pltpu.VMEM((DEG_CHUNKS, CHUNK), jnp.int32),
        pltpu.VMEM((128,), jnp.float32),
        pltpu.VMEM_SHARED((NPD,), jnp.float32),
        pltpu.SemaphoreType.DMA,
    ],
)
def _deg_kernel(dst_hbm, zeros1_hbm, degp_hbm, dstv, ones_v, deg_sp, dsem):
    cid = lax.axis_index("c")
    sid = lax.axis_index("s")
    wid = cid * NS + sid
    pltpu.sync_copy(dst_hbm.at[pl.ds(wid * DEG_CHUNKS, DEG_CHUNKS)], dstv)
    pltpu.sync_copy(
        zeros1_hbm.at[pl.ds(sid * DROWS_PER_TILE, DROWS_PER_TILE)],
        deg_sp.at[pl.ds(sid * DROWS_PER_TILE, DROWS_PER_TILE)],
    )
    for k in range(128 // 16):
        ones_v[pl.ds(k * 16, 16)] = jnp.full((16,), 1.0, jnp.float32)
    plsc.subcore_barrier()

    def body(j, carry):
        pltpu.async_copy(ones_v.at[pl.ds(0, CHUNK)], deg_sp.at[dstv.at[j]],
                         dsem, add=True)
        return carry

    lax.fori_loop(0, DEG_CHUNKS, body, 0)

    def drain(j, carry):
        pltpu.make_async_copy(ones_v.at[pl.ds(0, CHUNK)],
                              deg_sp.at[dstv.at[j]], dsem).wait()
        return carry

    lax.fori_loop(0, DEG_CHUNKS, drain, 0)
    plsc.subcore_barrier()
    pltpu.sync_copy(
        deg_sp.at[pl.ds(sid * DROWS_PER_TILE, DROWS_PER_TILE)],
        degp_hbm.at[cid, pl.ds(sid * DROWS_PER_TILE, DROWS_PER_TILE)],
    )


@functools.partial(
    pl.kernel,
    out_type=jax.ShapeDtypeStruct((NC, N_NODES, D), jnp.float32),
    mesh=_mesh,
    scratch_types=[
        pltpu.VMEM((RING, CHUNK), jnp.int32),
        pltpu.VMEM((RING, CHUNK), jnp.int32),
        pltpu.VMEM((CHUNK, D), jnp.float32),
        pltpu.VMEM((CHUNK, D), jnp.float32),
        pltpu.VMEM_SHARED((N_NODES, D), jnp.float32),
        pltpu.SemaphoreType.DMA,
        pltpu.SemaphoreType.DMA,
    ],
)
def _msg_kernel(src_hbm, dst_hbm, g2_hbm, zeros2_hbm, outp_hbm,
                srcv, dstv, buf0, buf1, acc_sp, sem0, sem1):
    cid = lax.axis_index("c")
    sid = lax.axis_index("s")
    g_hbm = g2_hbm
    chunk0 = (cid * NS + sid) * TILE_CHUNKS
    @pl.when(sid < NS - 1)
    def _():
        pltpu.sync_copy(zeros2_hbm.at[pl.ds(sid * RPT, RPT)],
                        acc_sp.at[pl.ds(sid * RPT, RPT)])

    @pl.when(sid == NS - 1)
    def _():
        pltpu.sync_copy(zeros2_hbm.at[pl.ds((NS - 1) * RPT, N_NODES - (NS - 1) * RPT)],
                        acc_sp.at[pl.ds((NS - 1) * RPT, N_NODES - (NS - 1) * RPT)])

    plsc.subcore_barrier()

    def ring_body(s, carry):
        base = chunk0 + s * RING
        pltpu.sync_copy(src_hbm.at[pl.ds(base, RING)], srcv)
        pltpu.sync_copy(dst_hbm.at[pl.ds(base, RING)], dstv)
        pltpu.make_async_copy(g_hbm.at[srcv.at[0]], buf0, sem0).start()

        def body(j, inner):
            c0 = 2 * j
            c1 = 2 * j + 1
            pltpu.make_async_copy(g_hbm.at[srcv.at[c1]], buf1, sem1).start()
            pltpu.make_async_copy(g_hbm.at[srcv.at[c0]], buf0, sem0).wait()
            pltpu.sync_copy(buf0, acc_sp.at[dstv.at[c0]], add=True)

            @pl.when(c0 + 2 < RING)
            def _():
                pltpu.make_async_copy(g_hbm.at[srcv.at[c0 + 2]], buf0,
                                      sem0).start()

            pltpu.make_async_copy(g_hbm.at[srcv.at[c1]], buf1, sem1).wait()
            pltpu.sync_copy(buf1, acc_sp.at[dstv.at[c1]], add=True)
            return inner

        lax.fori_loop(0, RING // 2, body, 0)
        return carry

    lax.fori_loop(0, TILE_CHUNKS // RING, ring_body, 0)
    plsc.subcore_barrier()

    @pl.when(sid < NS - 1)
    def _():
        pltpu.sync_copy(acc_sp.at[pl.ds(sid * RPT, RPT)],
                        outp_hbm.at[cid, pl.ds(sid * RPT, RPT)])

    @pl.when(sid == NS - 1)
    def _():
        pltpu.sync_copy(acc_sp.at[pl.ds((NS - 1) * RPT, N_NODES - (NS - 1) * RPT)],
                        outp_hbm.at[cid, pl.ds((NS - 1) * RPT, N_NODES - (NS - 1) * RPT)])


_BLK = 2000
_GRID = N_NODES // _BLK


def _matmul_body(x_ref, w_ref, h_ref):
    h_ref[...] = jnp.dot(x_ref[...], w_ref[...],
                         preferred_element_type=jnp.float32)


def _matmul_call(x, w):
    return pl.pallas_call(
        _matmul_body,
        grid=(_GRID,),
        in_specs=[
            pl.BlockSpec((_BLK, D), lambda i: (i, 0)),
            pl.BlockSpec((D, D), lambda i: (0, 0)),
        ],
        out_specs=pl.BlockSpec((_BLK, D), lambda i: (i, 0)),
        out_shape=jax.ShapeDtypeStruct((N_NODES, D), jnp.float32),
    )(x, w)


def _scale_body(h_ref, d0_ref, d1_ref, g2_ref, dinv_ref):
    deg = d0_ref[...] + d1_ref[...] + 1.0
    dinv = lax.rsqrt(deg)
    g2_ref[...] = h_ref[...] * dinv
    dinv_ref[...] = dinv


def _scale_call(h, d0, d1):
    return pl.pallas_call(
        _scale_body,
        grid=(_GRID,),
        in_specs=[
            pl.BlockSpec((_BLK, D), lambda i: (i, 0)),
            pl.BlockSpec((_BLK, 1), lambda i: (i, 0)),
            pl.BlockSpec((_BLK, 1), lambda i: (i, 0)),
        ],
        out_specs=[
            pl.BlockSpec((_BLK, D), lambda i: (i, 0)),
            pl.BlockSpec((_BLK, 1), lambda i: (i, 0)),
        ],
        out_shape=[
            jax.ShapeDtypeStruct((N_NODES, D), jnp.float32),
            jax.ShapeDtypeStruct((N_NODES, 1), jnp.float32),
        ],
    )(h, d0, d1)


def _combine_body(p_ref, g_ref, dinv_ref, b_ref, o_ref):
    s = (p_ref[0] + p_ref[1] + g_ref[...]) * dinv_ref[...]
    o_ref[...] = jnp.maximum(s + b_ref[...], 0.0)


def _combine_call(partials, g, dinv, b2):
    return pl.pallas_call(
        _combine_body,
        grid=(_GRID,),
        in_specs=[
            pl.BlockSpec((NC, _BLK, D), lambda i: (0, i, 0)),
            pl.BlockSpec((_BLK, D), lambda i: (i, 0)),
            pl.BlockSpec((_BLK, 1), lambda i: (i, 0)),
            pl.BlockSpec((1, D), lambda i: (0, 0)),
        ],
        out_specs=pl.BlockSpec((_BLK, D), lambda i: (i, 0)),
        out_shape=jax.ShapeDtypeStruct((N_NODES, D), jnp.float32),
    )(partials, g, dinv, b2)


def kernel(x, edge_index, W, b):
    src2 = edge_index[0].astype(jnp.int32).reshape(TOT_CHUNKS, CHUNK)
    dst2 = edge_index[1].astype(jnp.int32).reshape(TOT_CHUNKS, CHUNK)

    zeros1 = jnp.zeros((NPD,), jnp.float32)
    zeros2 = jnp.zeros((N_NODES, D), jnp.float32)

    degp = _deg_kernel(dst2, zeros1)
    h = _matmul_call(x, W)
    d0 = degp[0, :N_NODES].reshape(N_NODES, 1)
    d1 = degp[1, :N_NODES].reshape(N_NODES, 1)
    g2, dinv = _scale_call(h, d0, d1)

    partials = _msg_kernel(src2, dst2, g2, zeros2)

    return _combine_call(partials, g2, dinv, b.reshape(1, D))


# speedup vs baseline: 1.0647x; 1.0150x over previous
import functools

import jax
import jax.numpy as jnp
from jax import lax
from jax.experimental import pallas as pl
from jax.experimental.pallas import tpu as pltpu
from jax.experimental.pallas import tpu_sc as plsc

N_NODES = 10000
D = 128
N_EDGES = 320000

NC = 2
NS = 16
NPD = 10240
CHUNK = 125
TOT_CHUNKS = 2560
RING = 40
TILE_CHUNKS = TOT_CHUNKS // (NC * NS)
DEG_CHUNKS = TILE_CHUNKS
RPT = 632
DROWS_PER_TILE = NPD // NS

_mesh = plsc.VectorSubcoreMesh(core_axis_name="c", subcore_axis_name="s")


@functools.partial(
    pl.kernel,
    out_type=jax.ShapeDtypeStruct((NC, NPD), jnp.float32),
    mesh=_mesh,
    scratch_types=[
        pltpu.VMEM((DEG_CHUNKS, CHUNK), jnp.int32),
        pltpu.VMEM((128,), jnp.float32),
        pltpu.VMEM_SHARED((NPD,), jnp.float32),
        pltpu.SemaphoreType.DMA,
    ],
)
def _deg_kernel(dst_hbm, zeros1_hbm, degp_hbm, dstv, ones_v, deg_sp, dsem):
    cid = lax.axis_index("c")
    sid = lax.axis_index("s")
    wid = cid * NS + sid
    pltpu.sync_copy(dst_hbm.at[pl.ds(wid * DEG_CHUNKS, DEG_CHUNKS)], dstv)
    pltpu.sync_copy(
        zeros1_hbm.at[pl.ds(sid * DROWS_PER_TILE, DROWS_PER_TILE)],
        deg_sp.at[pl.ds(sid * DROWS_PER_TILE, DROWS_PER_TILE)],
    )
    for k in range(128 // 16):
        ones_v[pl.ds(k * 16, 16)] = jnp.full((16,), 1.0, jnp.float32)
    plsc.subcore_barrier()

    def body(j, carry):
        pltpu.async_copy(ones_v.at[pl.ds(0, CHUNK)], deg_sp.at[dstv.at[j]],
                         dsem, add=True)
        return carry

    lax.fori_loop(0, DEG_CHUNKS, body, 0)

    def drain(j, carry):
        pltpu.make_async_copy(ones_v.at[pl.ds(0, CHUNK)],
                              deg_sp.at[dstv.at[j]], dsem).wait()
        return carry

    lax.fori_loop(0, DEG_CHUNKS, drain, 0)
    plsc.subcore_barrier()
    pltpu.sync_copy(
        deg_sp.at[pl.ds(sid * DROWS_PER_TILE, DROWS_PER_TILE)],
        degp_hbm.at[cid, pl.ds(sid * DROWS_PER_TILE, DROWS_PER_TILE)],
    )


@functools.partial(
    pl.kernel,
    out_type=jax.ShapeDtypeStruct((NC, N_NODES, D), jnp.float32),
    mesh=_mesh,
    scratch_types=[
        pltpu.VMEM((RING, CHUNK), jnp.int32),
        pltpu.VMEM((RING, CHUNK), jnp.int32),
        pltpu.VMEM((CHUNK, D), jnp.float32),
        pltpu.VMEM((CHUNK, D), jnp.float32),
        pltpu.VMEM_SHARED((N_NODES, D), jnp.float32),
        pltpu.SemaphoreType.DMA,
        pltpu.SemaphoreType.DMA,
    ],
)
def _msg_kernel(src_hbm, dst_hbm, g2_hbm, zeros2_hbm, outp_hbm,
                srcv, dstv, buf0, buf1, acc_sp, sem0, sem1):
    cid = lax.axis_index("c")
    sid = lax.axis_index("s")
    g_hbm = g2_hbm
    chunk0 = (cid * NS + sid) * TILE_CHUNKS
    @pl.when(sid < NS - 1)
    def _():
        pltpu.sync_copy(zeros2_hbm.at[pl.ds(sid * RPT, RPT)],
                        acc_sp.at[pl.ds(sid * RPT, RPT)])

    @pl.when(sid == NS - 1)
    def _():
        pltpu.sync_copy(zeros2_hbm.at[pl.ds((NS - 1) * RPT, N_NODES - (NS - 1) * RPT)],
                        acc_sp.at[pl.ds((NS - 1) * RPT, N_NODES - (NS - 1) * RPT)])

    plsc.subcore_barrier()

    def ring_body(s, carry):
        base = chunk0 + s * RING
        pltpu.sync_copy(src_hbm.at[pl.ds(base, RING)], srcv)
        pltpu.sync_copy(dst_hbm.at[pl.ds(base, RING)], dstv)
        pltpu.make_async_copy(g_hbm.at[srcv.at[0]], buf0, sem0).start()

        def body(j, inner):
            c0 = 2 * j
            c1 = 2 * j + 1
            pltpu.make_async_copy(g_hbm.at[srcv.at[c1]], buf1, sem1).start()
            pltpu.make_async_copy(g_hbm.at[srcv.at[c0]], buf0, sem0).wait()
            pltpu.sync_copy(buf0, acc_sp.at[dstv.at[c0]], add=True)

            @pl.when(c0 + 2 < RING)
            def _():
                pltpu.make_async_copy(g_hbm.at[srcv.at[c0 + 2]], buf0,
                                      sem0).start()

            pltpu.make_async_copy(g_hbm.at[srcv.at[c1]], buf1, sem1).wait()
            pltpu.sync_copy(buf1, acc_sp.at[dstv.at[c1]], add=True)
            return inner

        lax.fori_loop(0, RING // 2, body, 0)
        return carry

    lax.fori_loop(0, TILE_CHUNKS // RING, ring_body, 0)
    plsc.subcore_barrier()

    @pl.when(sid < NS - 1)
    def _():
        pltpu.sync_copy(acc_sp.at[pl.ds(sid * RPT, RPT)],
                        outp_hbm.at[cid, pl.ds(sid * RPT, RPT)])

    @pl.when(sid == NS - 1)
    def _():
        pltpu.sync_copy(acc_sp.at[pl.ds((NS - 1) * RPT, N_NODES - (NS - 1) * RPT)],
                        outp_hbm.at[cid, pl.ds((NS - 1) * RPT, N_NODES - (NS - 1) * RPT)])


_BLK = 2000
_GRID = N_NODES // _BLK


def _scale_body(x_ref, w_ref, d0_ref, d1_ref, g2_ref, dinv_ref):
    deg = d0_ref[...] + d1_ref[...] + 1.0
    dinv = lax.rsqrt(deg)
    h = jnp.dot(x_ref[...], w_ref[...], preferred_element_type=jnp.float32)
    g2_ref[...] = h * dinv
    dinv_ref[...] = dinv


def _scale_call(x, w, d0, d1):
    return pl.pallas_call(
        _scale_body,
        grid=(_GRID,),
        in_specs=[
            pl.BlockSpec((_BLK, D), lambda i: (i, 0)),
            pl.BlockSpec((D, D), lambda i: (0, 0)),
            pl.BlockSpec((_BLK, 1), lambda i: (i, 0)),
            pl.BlockSpec((_BLK, 1), lambda i: (i, 0)),
        ],
        out_specs=[
            pl.BlockSpec((_BLK, D), lambda i: (i, 0)),
            pl.BlockSpec((_BLK, 1), lambda i: (i, 0)),
        ],
        out_shape=[
            jax.ShapeDtypeStruct((N_NODES, D), jnp.float32),
            jax.ShapeDtypeStruct((N_NODES, 1), jnp.float32),
        ],
    )(x, w, d0, d1)


def _combine_body(p_ref, g_ref, dinv_ref, b_ref, o_ref):
    s = (p_ref[0] + p_ref[1] + g_ref[...]) * dinv_ref[...]
    o_ref[...] = jnp.maximum(s + b_ref[...], 0.0)


def _combine_call(partials, g, dinv, b2):
    return pl.pallas_call(
        _combine_body,
        grid=(_GRID,),
        in_specs=[
            pl.BlockSpec((NC, _BLK, D), lambda i: (0, i, 0)),
            pl.BlockSpec((_BLK, D), lambda i: (i, 0)),
            pl.BlockSpec((_BLK, 1), lambda i: (i, 0)),
            pl.BlockSpec((1, D), lambda i: (0, 0)),
        ],
        out_specs=pl.BlockSpec((_BLK, D), lambda i: (i, 0)),
        out_shape=jax.ShapeDtypeStruct((N_NODES, D), jnp.float32),
    )(partials, g, dinv, b2)


def kernel(x, edge_index, W, b):
    src2 = edge_index[0].astype(jnp.int32).reshape(TOT_CHUNKS, CHUNK)
    dst2 = edge_index[1].astype(jnp.int32).reshape(TOT_CHUNKS, CHUNK)

    zeros1 = jnp.zeros((NPD,), jnp.float32)
    zeros2 = jnp.zeros((N_NODES, D), jnp.float32)

    degp = _deg_kernel(dst2, zeros1)
    d0 = degp[0, :N_NODES].reshape(N_NODES, 1)
    d1 = degp[1, :N_NODES].reshape(N_NODES, 1)
    g2, dinv = _scale_call(x, W, d0, d1)

    partials = _msg_kernel(src2, dst2, g2, zeros2)

    return _combine_call(partials, g2, dinv, b.reshape(1, D))
